# Initial kernel scaffold; baseline (speedup 1.0000x reference)
import jax
import jax.numpy as jnp
from jax.experimental import pallas as pl


def kernel(h, edge_index, edge_attr, W1, b1, W2, b2, U1, bu1, U2, bu2):
    raise NotImplementedError("write your pallas kernel here")


# speedup vs baseline: 3.6688x; 3.6688x over previous
import functools

import jax
import jax.numpy as jnp
from jax import lax
from jax.experimental import pallas as pl
from jax.experimental.pallas import tpu as pltpu
from jax.experimental.pallas import tpu_sc as plsc

N, E, D, DE = 10000, 320000, 128, 16

NC, NS = 2, 16
NW = NC * NS
EPW = E // NW
CHUNK = 80
NCHUNK = EPW // CHUNK
ROW_TILE = 624

_sc_mesh = plsc.VectorSubcoreMesh(core_axis_name="c", subcore_axis_name="s")


def _premix_body(h_ref, w1a_ref, w1b_ref, b1_ref, ha_ref, hb_ref):
    h = h_ref[...]
    ha_ref[...] = jnp.dot(h, w1a_ref[...], preferred_element_type=jnp.float32)
    hb_ref[...] = (jnp.dot(h, w1b_ref[...], preferred_element_type=jnp.float32)
                   + b1_ref[...])


def _edge_body(ga_ref, gb_ref, ea_ref, w1c_ref, w2_ref, b2_ref, m_ref):
    pre = (ga_ref[...] + gb_ref[...]
           + jnp.dot(ea_ref[...], w1c_ref[...],
                     preferred_element_type=jnp.float32))
    m1 = jnp.maximum(pre, 0.0)
    m_ref[...] = jnp.maximum(
        jnp.dot(m1, w2_ref[...], preferred_element_type=jnp.float32)
        + b2_ref[...], 0.0)


def _node_body(h_ref, p0_ref, p1_ref, u1a_ref, u1b_ref, bu1_ref,
               u2_ref, bu2_ref, o_ref):
    aggr = p0_ref[...] + p1_ref[...]
    t = jnp.maximum(
        jnp.dot(h_ref[...], u1a_ref[...], preferred_element_type=jnp.float32)
        + jnp.dot(aggr, u1b_ref[...], preferred_element_type=jnp.float32)
        + bu1_ref[...], 0.0)
    o_ref[...] = jnp.maximum(
        jnp.dot(t, u2_ref[...], preferred_element_type=jnp.float32)
        + bu2_ref[...], 0.0)


def _rep(shape):
    return pl.BlockSpec(shape, lambda i: (0,) * len(shape))


def _premix(h, w1a, w1b, b1r):
    blk = 1000
    return pl.pallas_call(
        _premix_body,
        grid=(N // blk,),
        in_specs=[pl.BlockSpec((blk, D), lambda i: (i, 0)),
                  _rep((D, D)), _rep((D, D)), _rep((1, D))],
        out_specs=[pl.BlockSpec((blk, D), lambda i: (i, 0)),
                   pl.BlockSpec((blk, D), lambda i: (i, 0))],
        out_shape=[jax.ShapeDtypeStruct((N, D), jnp.float32),
                   jax.ShapeDtypeStruct((N, D), jnp.float32)],
    )(h, w1a, w1b, b1r)


def _edge_mlp(ga, gb, ea, w1c, w2, b2r):
    blk = 3200
    return pl.pallas_call(
        _edge_body,
        grid=(E // blk,),
        in_specs=[pl.BlockSpec((blk, D), lambda i: (i, 0)),
                  pl.BlockSpec((blk, D), lambda i: (i, 0)),
                  pl.BlockSpec((blk, DE), lambda i: (i, 0)),
                  _rep((DE, D)), _rep((D, D)), _rep((1, D))],
        out_specs=pl.BlockSpec((blk, D), lambda i: (i, 0)),
        out_shape=jax.ShapeDtypeStruct((E, D), jnp.float32),
    )(ga, gb, ea, w1c, w2, b2r)


def _node_mlp(h, p0, p1, u1a, u1b, bu1r, u2, bu2r):
    blk = 1000
    return pl.pallas_call(
        _node_body,
        grid=(N // blk,),
        in_specs=[pl.BlockSpec((blk, D), lambda i: (i, 0)),
                  pl.BlockSpec((blk, D), lambda i: (i, 0)),
                  pl.BlockSpec((blk, D), lambda i: (i, 0)),
                  _rep((D, D)), _rep((D, D)), _rep((1, D)),
                  _rep((D, D)), _rep((1, D))],
        out_specs=pl.BlockSpec((blk, D), lambda i: (i, 0)),
        out_shape=jax.ShapeDtypeStruct((N, D), jnp.float32),
    )(h, p0, p1, u1a, u1b, bu1r, u2, bu2r)


@functools.partial(
    pl.kernel,
    mesh=_sc_mesh,
    out_type=(jax.ShapeDtypeStruct((E, D), jnp.float32),
              jax.ShapeDtypeStruct((E, D), jnp.float32)),
    scratch_types=[
        pltpu.VMEM((NCHUNK, CHUNK), jnp.int32),
        pltpu.VMEM((NCHUNK, CHUNK), jnp.int32),
        pltpu.VMEM((CHUNK, D), jnp.float32),
        pltpu.VMEM((CHUNK, D), jnp.float32),
        pltpu.SemaphoreType.DMA,
        pltpu.SemaphoreType.DMA,
    ],
)
def _sc_gather(dst_hbm, src_hbm, ha_hbm, hb_hbm, ga_hbm, gb_hbm,
               dsti_v, srci_v, rowsa_v, rowsb_v, sema, semb):
    c = lax.axis_index("c")
    s = lax.axis_index("s")
    w = c * NS + s
    base = w * EPW
    pltpu.sync_copy(dst_hbm.at[w], dsti_v)
    pltpu.sync_copy(src_hbm.at[w], srci_v)

    def step(j, carry):
        cpa = pltpu.async_copy(ha_hbm.at[dsti_v.at[j]], rowsa_v, sema)
        cpb = pltpu.async_copy(hb_hbm.at[srci_v.at[j]], rowsb_v, semb)
        cpa.wait()
        cpb.wait()
        pltpu.sync_copy(rowsa_v, ga_hbm.at[pl.ds(base + j * CHUNK, CHUNK)])
        pltpu.sync_copy(rowsb_v, gb_hbm.at[pl.ds(base + j * CHUNK, CHUNK)])
        return carry

    lax.fori_loop(0, NCHUNK, step, 0)


@functools.partial(
    pl.kernel,
    mesh=_sc_mesh,
    out_type=jax.ShapeDtypeStruct((NC, N, D), jnp.float32),
    scratch_types=[
        pltpu.VMEM((NCHUNK, CHUNK), jnp.int32),
        pltpu.VMEM((CHUNK, D), jnp.float32),
        pltpu.VMEM_SHARED((N, D), jnp.float32),
        pltpu.SemaphoreType.DMA,
    ],
)
def _sc_scatter(dst_hbm, m_hbm, zeros_hbm, out_hbm,
                dsti_v, rows_v, accum_sh, sem):
    c = lax.axis_index("c")
    s = lax.axis_index("s")
    w = c * NS + s
    base = w * EPW
    r0 = s * ROW_TILE
    pltpu.sync_copy(zeros_hbm.at[pl.ds(r0, ROW_TILE)],
                    accum_sh.at[pl.ds(r0, ROW_TILE)])

    @pl.when(s == NS - 1)
    def _():
        pltpu.sync_copy(zeros_hbm.at[pl.ds(NS * ROW_TILE, N - NS * ROW_TILE)],
                        accum_sh.at[pl.ds(NS * ROW_TILE, N - NS * ROW_TILE)])

    pltpu.sync_copy(dst_hbm.at[w], dsti_v)
    plsc.subcore_barrier()

    def step(j, carry):
        pltpu.sync_copy(m_hbm.at[pl.ds(base + j * CHUNK, CHUNK)], rows_v)
        pltpu.sync_copy(rows_v, accum_sh.at[dsti_v.at[j]], add=True)
        return carry

    lax.fori_loop(0, NCHUNK, step, 0)
    plsc.subcore_barrier()
    pltpu.sync_copy(accum_sh.at[pl.ds(r0, ROW_TILE)],
                    out_hbm.at[c, pl.ds(r0, ROW_TILE)])

    @pl.when(s == NS - 1)
    def _():
        pltpu.sync_copy(accum_sh.at[pl.ds(NS * ROW_TILE, N - NS * ROW_TILE)],
                        out_hbm.at[c, pl.ds(NS * ROW_TILE, N - NS * ROW_TILE)])


def kernel(h, edge_index, edge_attr, W1, b1, W2, b2, U1, bu1, U2, bu2):
    w1a, w1b, w1c = W1[:D], W1[D:2 * D], W1[2 * D:]
    u1a, u1b = U1[:D], U1[D:]
    b1r, b2r = b1.reshape(1, D), b2.reshape(1, D)
    bu1r, bu2r = bu1.reshape(1, D), bu2.reshape(1, D)
    src3 = edge_index[0].reshape(NW, NCHUNK, CHUNK)
    dst3 = edge_index[1].reshape(NW, NCHUNK, CHUNK)

    ha, hb = _premix(h, w1a, w1b, b1r)
    ga, gb = _sc_gather(dst3, src3, ha, hb)
    m = _edge_mlp(ga, gb, edge_attr, w1c, w2=W2, b2r=b2r)
    parts = _sc_scatter(dst3, m, jnp.zeros((N, D), jnp.float32))
    return _node_mlp(h, parts[0], parts[1], u1a, u1b, bu1r, U2, bu2r)
